# baseline (speedup 1.0000x reference)
import functools

import jax
import jax.numpy as jnp
from jax import lax
from jax.experimental import pallas as pl
from jax.experimental.pallas import tpu as pltpu
from jax.experimental.pallas import tpu_sc as plsc

N_ATOMS = 2_000_000
NUM_SPECIES = 119
TABLE_PAD = 128

NC, NS, L = 2, 16, 16
NW = NC * NS

UNROLL = 4
BLK = UNROLL * L
CHUNK = 62_464
NCHUNK = 4
CSZ = CHUNK // NCHUNK
TAIL_OFF = NW * CHUNK
TAIL = N_ATOMS - TAIL_OFF
TAIL_TILES = TAIL // BLK
HIST = L * TABLE_PAD


def _hist_loop(idx_ref, hist_ref, n_iters, lane_off):
    ones = jnp.ones((L,), jnp.int32)

    def step(i, carry):
        base = i * BLK
        for u in range(UNROLL):
            idx = idx_ref[pl.ds(base + u * L, L)]
            plsc.addupdate_scatter(hist_ref, [idx + lane_off], ones)
        return carry

    return lax.fori_loop(0, n_iters, step, 0)


def _sc_partials(body):
    return pl.kernel(
        body,
        out_type=jax.ShapeDtypeStruct((NW, L), jnp.float32),
        mesh=plsc.VectorSubcoreMesh(core_axis_name="c", subcore_axis_name="s"),
        scratch_types=[
            pltpu.VMEM((CSZ,), jnp.int32),
            pltpu.VMEM((CSZ,), jnp.int32),
            pltpu.VMEM((HIST,), jnp.int32),
            pltpu.VMEM((TABLE_PAD,), jnp.float32),
            pltpu.VMEM((BLK,), jnp.int32),
            pltpu.VMEM((L,), jnp.float32),
            pltpu.SemaphoreType.DMA,
            pltpu.SemaphoreType.DMA,
        ],
        compiler_params=pltpu.CompilerParams(needs_layout_passes=False),
    )


@_sc_partials
def _lookup_sum_body(idx_hbm, tbl_hbm, out_hbm, buf0, buf1, hist_v, tbl_v,
                     tail_v, acc_v, sem0, sem1):
    wid = lax.axis_index("s") * NC + lax.axis_index("c")
    base = wid * CHUNK
    bufs = (buf0, buf1)
    sems = (sem0, sem1)

    copies = [pltpu.async_copy(idx_hbm.at[pl.ds(base, CSZ)], buf0, sem0)]

    zf = jnp.zeros((L,), jnp.float32)
    tbl_v[pl.ds(TABLE_PAD - L, L)] = zf
    pltpu.sync_copy(tbl_hbm, tbl_v.at[pl.ds(0, NUM_SPECIES)])

    zi = jnp.zeros((L,), jnp.int32)

    def zstep(i, carry):
        hist_ref_slice = pl.ds(i * L, L)
        hist_v[hist_ref_slice] = zi
        return carry

    lax.fori_loop(0, HIST // L, zstep, 0)

    lane_off = lax.iota(jnp.int32, L) * TABLE_PAD

    for t in range(NCHUNK):
        if t + 1 < NCHUNK:
            copies.append(
                pltpu.async_copy(
                    idx_hbm.at[pl.ds(base + (t + 1) * CSZ, CSZ)],
                    bufs[(t + 1) % 2], sems[(t + 1) % 2]))
        copies[t].wait()
        _hist_loop(bufs[t % 2], hist_v, CSZ // BLK, lane_off)

    @pl.when(wid < TAIL_TILES)
    def _():
        pltpu.sync_copy(idx_hbm.at[pl.ds(TAIL_OFF + wid * BLK, BLK)], tail_v)
        _hist_loop(tail_v, hist_v, 1, lane_off)

    def dstep(r, acc):
        for s8 in range(TABLE_PAD // L):
            c = hist_v[pl.ds(r * TABLE_PAD + s8 * L, L)]
            e = tbl_v[pl.ds(s8 * L, L)]
            acc = acc + c.astype(jnp.float32) * e
        return acc

    acc_v[...] = lax.fori_loop(0, L, dstep, jnp.zeros((L,), jnp.float32))
    pltpu.sync_copy(acc_v, out_hbm.at[wid])


def kernel(atomic_numbers, atomic_energies, z_keys):
    del z_keys
    partials = _lookup_sum_body(atomic_numbers, atomic_energies)
    return jnp.sum(partials)


# speedup vs baseline: 1.7058x; 1.7058x over previous
import functools

import jax
import jax.numpy as jnp
from jax import lax
from jax.experimental import pallas as pl
from jax.experimental.pallas import tpu as pltpu
from jax.experimental.pallas import tpu_sc as plsc

N_ATOMS = 2_000_000
NUM_SPECIES = 119
TABLE_PAD = 128

NC, NS, L = 2, 16, 16
NW = NC * NS

UNROLL = 4
BLK = UNROLL * L
CHUNK = 62_464
NCHUNK = 4
CSZ = CHUNK // NCHUNK
TAIL_OFF = NW * CHUNK
TAIL = N_ATOMS - TAIL_OFF
TAIL_TILES = TAIL // BLK


def _gather_sum_loop(idx_ref, tbl_ref, n_iters, accs):
    @plsc.parallel_loop(0, n_iters, step=1, unroll=1, carry=accs)
    def step(i, carry):
        base = i * BLK
        out = []
        for u in range(UNROLL):
            idx = idx_ref[pl.ds(base + u * L, L)]
            vals = plsc.load_gather(tbl_ref, [idx])
            out.append(carry[u] + vals)
        return tuple(out)

    return step


def _sc_partials(body):
    return pl.kernel(
        body,
        out_type=jax.ShapeDtypeStruct((NW, L), jnp.float32),
        mesh=plsc.VectorSubcoreMesh(core_axis_name="c", subcore_axis_name="s"),
        scratch_types=[
            pltpu.VMEM((CSZ,), jnp.int32),
            pltpu.VMEM((CSZ,), jnp.int32),
            pltpu.VMEM((CSZ,), jnp.int32),
            pltpu.VMEM((CSZ,), jnp.int32),
            pltpu.VMEM((TABLE_PAD,), jnp.float32),
            pltpu.VMEM((BLK,), jnp.int32),
            pltpu.VMEM((L,), jnp.float32),
            pltpu.SemaphoreType.DMA,
            pltpu.SemaphoreType.DMA,
            pltpu.SemaphoreType.DMA,
            pltpu.SemaphoreType.DMA,
        ],
        compiler_params=pltpu.CompilerParams(needs_layout_passes=False),
    )


@_sc_partials
def _lookup_sum_body(idx_hbm, tbl_hbm, out_hbm, buf0, buf1, buf2, buf3,
                     tbl_v, tail_v, acc_v, sem0, sem1, sem2, sem3):
    wid = lax.axis_index("s") * NC + lax.axis_index("c")
    base = wid * CHUNK
    bufs = (buf0, buf1, buf2, buf3)
    sems = (sem0, sem1, sem2, sem3)

    copies = [
        pltpu.async_copy(idx_hbm.at[pl.ds(base + t * CSZ, CSZ)], bufs[t],
                         sems[t]) for t in range(NCHUNK)
    ]
    pltpu.sync_copy(tbl_hbm, tbl_v.at[pl.ds(0, NUM_SPECIES)])

    zeros = jnp.zeros((L,), jnp.float32)
    accs = (zeros, zeros, zeros, zeros)
    for t in range(NCHUNK):
        copies[t].wait()
        accs = _gather_sum_loop(bufs[t], tbl_v, CSZ // BLK, accs)
    acc_v[...] = (accs[0] + accs[1]) + (accs[2] + accs[3])

    @pl.when(wid < TAIL_TILES)
    def _():
        pltpu.sync_copy(idx_hbm.at[pl.ds(TAIL_OFF + wid * BLK, BLK)], tail_v)
        a = acc_v[...]
        for u in range(UNROLL):
            idx = tail_v[pl.ds(u * L, L)]
            a = a + plsc.load_gather(tbl_v, [idx])
        acc_v[...] = a

    pltpu.sync_copy(acc_v, out_hbm.at[wid])


def kernel(atomic_numbers, atomic_energies, z_keys):
    del z_keys
    partials = _lookup_sum_body(atomic_numbers, atomic_energies)
    return jnp.sum(partials)


# speedup vs baseline: 1.7805x; 1.0438x over previous
import functools

import jax
import jax.numpy as jnp
from jax import lax
from jax.experimental import pallas as pl
from jax.experimental.pallas import tpu as pltpu
from jax.experimental.pallas import tpu_sc as plsc

N_ATOMS = 2_000_000
NUM_SPECIES = 119
TABLE_PAD = 128

NC, NS, L = 2, 16, 16
NW = NC * NS

UNROLL = 4
BLK = UNROLL * L
CHUNK = 62_464
NCHUNK = 4
CSZ = CHUNK // NCHUNK
TAIL_OFF = NW * CHUNK
TAIL = N_ATOMS - TAIL_OFF
TAIL_TILES = TAIL // BLK


def _gather_sum_loop(idx_ref, tbl_ref, n_iters, accs):
    @plsc.parallel_loop(0, n_iters, step=1, unroll=1, carry=accs)
    def step(i, carry):
        base = i * BLK
        out = []
        for u in range(UNROLL):
            idx = idx_ref[pl.ds(base + u * L, L)]
            vals = plsc.load_gather(tbl_ref, [idx])
            out.append(carry[u] + vals)
        return tuple(out)

    return step


def _sc_partials(body):
    return pl.kernel(
        body,
        out_type=jax.ShapeDtypeStruct((NW, L), jnp.float32),
        mesh=plsc.VectorSubcoreMesh(core_axis_name="c", subcore_axis_name="s"),
        scratch_types=[
            pltpu.VMEM((CSZ,), jnp.int32),
            pltpu.VMEM((CSZ,), jnp.int32),
            pltpu.VMEM((TABLE_PAD,), jnp.float32),
            pltpu.VMEM((BLK,), jnp.int32),
            pltpu.VMEM((L,), jnp.float32),
            pltpu.SemaphoreType.DMA,
            pltpu.SemaphoreType.DMA,
        ],
        compiler_params=pltpu.CompilerParams(needs_layout_passes=False),
    )


@_sc_partials
def _lookup_sum_body(idx_hbm, tbl_hbm, out_hbm, buf0, buf1, tbl_v, tail_v,
                     acc_v, sem0, sem1):
    wid = lax.axis_index("s") * NC + lax.axis_index("c")
    base = wid * CHUNK
    bufs = (buf0, buf1)
    sems = (sem0, sem1)

    sizes = (CSZ // 2, CSZ // 2, CSZ, CSZ, CSZ)
    offs = (0, CSZ // 2, CSZ, 2 * CSZ, 3 * CSZ)

    copies = [
        pltpu.async_copy(idx_hbm.at[pl.ds(base + offs[0], sizes[0])],
                         bufs[0].at[pl.ds(0, sizes[0])], sems[0])
    ]
    pltpu.sync_copy(tbl_hbm, tbl_v.at[pl.ds(0, NUM_SPECIES)])

    zeros = jnp.zeros((L,), jnp.float32)
    accs = (zeros, zeros, zeros, zeros)
    for t in range(len(sizes)):
        if t + 1 < len(sizes):
            b = (t + 1) % 2
            copies.append(
                pltpu.async_copy(
                    idx_hbm.at[pl.ds(base + offs[t + 1], sizes[t + 1])],
                    bufs[b].at[pl.ds(0, sizes[t + 1])], sems[b]))
        copies[t].wait()
        accs = _gather_sum_loop(bufs[t % 2], tbl_v, sizes[t] // BLK, accs)
    acc_v[...] = (accs[0] + accs[1]) + (accs[2] + accs[3])

    @pl.when(wid < TAIL_TILES)
    def _():
        pltpu.sync_copy(idx_hbm.at[pl.ds(TAIL_OFF + wid * BLK, BLK)], tail_v)
        a = acc_v[...]
        for u in range(UNROLL):
            idx = tail_v[pl.ds(u * L, L)]
            a = a + plsc.load_gather(tbl_v, [idx])
        acc_v[...] = a

    pltpu.sync_copy(acc_v, out_hbm.at[wid])


def kernel(atomic_numbers, atomic_energies, z_keys):
    del z_keys
    partials = _lookup_sum_body(atomic_numbers, atomic_energies)
    return jnp.sum(partials)


# speedup vs baseline: 1.8081x; 1.0155x over previous
import functools

import jax
import jax.numpy as jnp
from jax import lax
from jax.experimental import pallas as pl
from jax.experimental.pallas import tpu as pltpu
from jax.experimental.pallas import tpu_sc as plsc

N_ATOMS = 2_000_000
NUM_SPECIES = 119
TABLE_PAD = 128

NC, NS, L = 2, 16, 16
NW = NC * NS

UNROLL = 8
BLK = UNROLL * L
CHUNK = 62_464
NCHUNK = 4
CSZ = CHUNK // NCHUNK
TAIL_OFF = NW * CHUNK
TAIL = N_ATOMS - TAIL_OFF
TAIL_TILES = TAIL // BLK


def _gather_sum_loop(idx_ref, tbl_ref, n_iters, accs):
    @plsc.parallel_loop(0, n_iters, step=1, unroll=1, carry=accs)
    def step(i, carry):
        base = i * BLK
        out = []
        for u in range(UNROLL):
            idx = idx_ref[pl.ds(base + u * L, L)]
            vals = plsc.load_gather(tbl_ref, [idx])
            out.append(carry[u] + vals)
        return tuple(out)

    return step


def _sc_partials(body):
    return pl.kernel(
        body,
        out_type=jax.ShapeDtypeStruct((NW, L), jnp.float32),
        mesh=plsc.VectorSubcoreMesh(core_axis_name="c", subcore_axis_name="s"),
        scratch_types=[
            pltpu.VMEM((CSZ,), jnp.int32),
            pltpu.VMEM((CSZ,), jnp.int32),
            pltpu.VMEM((TABLE_PAD,), jnp.float32),
            pltpu.VMEM((BLK,), jnp.int32),
            pltpu.VMEM((L,), jnp.float32),
            pltpu.SemaphoreType.DMA,
            pltpu.SemaphoreType.DMA,
        ],
        compiler_params=pltpu.CompilerParams(needs_layout_passes=False),
    )


@_sc_partials
def _lookup_sum_body(idx_hbm, tbl_hbm, out_hbm, buf0, buf1, tbl_v, tail_v,
                     acc_v, sem0, sem1):
    wid = lax.axis_index("s") * NC + lax.axis_index("c")
    base = wid * CHUNK
    bufs = (buf0, buf1)
    sems = (sem0, sem1)

    copies = [pltpu.async_copy(idx_hbm.at[pl.ds(base, CSZ)], buf0, sem0)]
    pltpu.sync_copy(tbl_hbm, tbl_v.at[pl.ds(0, NUM_SPECIES)])

    zeros = jnp.zeros((L,), jnp.float32)
    accs = (zeros,) * UNROLL
    for t in range(NCHUNK):
        if t + 1 < NCHUNK:
            copies.append(
                pltpu.async_copy(
                    idx_hbm.at[pl.ds(base + (t + 1) * CSZ, CSZ)],
                    bufs[(t + 1) % 2], sems[(t + 1) % 2]))
        copies[t].wait()
        accs = _gather_sum_loop(bufs[t % 2], tbl_v, CSZ // BLK, accs)
    total = accs[0]
    for a in accs[1:]:
        total = total + a
    acc_v[...] = total

    @pl.when(wid < TAIL_TILES)
    def _():
        pltpu.sync_copy(idx_hbm.at[pl.ds(TAIL_OFF + wid * BLK, BLK)], tail_v)
        a = acc_v[...]
        for u in range(UNROLL):
            idx = tail_v[pl.ds(u * L, L)]
            a = a + plsc.load_gather(tbl_v, [idx])
        acc_v[...] = a

    pltpu.sync_copy(acc_v, out_hbm.at[wid])


def kernel(atomic_numbers, atomic_energies, z_keys):
    del z_keys
    partials = _lookup_sum_body(atomic_numbers, atomic_energies)
    return jnp.sum(partials)
